# baseline (speedup 1.0000x reference)
import jax
import jax.numpy as jnp
from jax import lax
from jax.experimental import pallas as pl
from jax.experimental.pallas import tpu as pltpu
from jax.experimental.pallas import tpu_sc as plsc

L = 16
NC, NS = 2, 16
NW = NC * NS
ROWS, COLS, D = 16384, 200, 5
WI = ROWS // NW
BJ = 8
NCH = COLS // BJ


def _sc_lookup(xt_hbm, tbl_hbm, y_hbm, x_v, y_v, tbl_v):
    wid = lax.axis_index("s") * NC + lax.axis_index("c")
    i0 = wid * WI
    pltpu.sync_copy(tbl_hbm, tbl_v)
    t0 = [tbl_v[d] for d in range(D)]
    t1 = [tbl_v[D + d] for d in range(D)]

    def chunk_body(c, carry):
        j0 = c * BJ
        pltpu.sync_copy(xt_hbm.at[pl.ds(j0, BJ), pl.ds(i0, WI)], x_v)

        def vec_body(g, carry2):
            r = g // (WI // L)
            col = (g % (WI // L)) * L
            m = x_v[r, pl.ds(col, L)] == 0
            for d in range(D):
                y_v[d, r, pl.ds(col, L)] = jnp.where(m, t0[d], t1[d])
            return carry2

        lax.fori_loop(0, BJ * (WI // L), vec_body, 0, unroll=4)
        for d in range(D):
            pltpu.sync_copy(y_v.at[d],
                            y_hbm.at[d, pl.ds(j0, BJ), pl.ds(i0, WI)])
        return carry

    lax.fori_loop(0, NCH, chunk_body, 0)


def kernel(x, table):
    xt = x.T
    tbl_splat = jnp.broadcast_to(
        table.reshape(2 * D)[:, None], (2 * D, L)).astype(jnp.float32)
    mesh = plsc.VectorSubcoreMesh(core_axis_name="c", subcore_axis_name="s")
    y = pl.kernel(
        _sc_lookup,
        out_type=jax.ShapeDtypeStruct((D, COLS, ROWS), jnp.float32),
        mesh=mesh,
        compiler_params=pltpu.CompilerParams(needs_layout_passes=False),
        scratch_types=[
            pltpu.VMEM((BJ, WI), jnp.int32),
            pltpu.VMEM((D, BJ, WI), jnp.float32),
            pltpu.VMEM((2 * D, L), jnp.float32),
        ],
    )(xt, tbl_splat)
    return y.transpose(2, 1, 0)


# speedup vs baseline: 1.1946x; 1.1946x over previous
import jax
import jax.numpy as jnp
from jax import lax
from jax.experimental import pallas as pl
from jax.experimental.pallas import tpu as pltpu
from jax.experimental.pallas import tpu_sc as plsc

L = 16
NC, NS = 2, 16
NW = NC * NS
ROWS, COLS, D = 16384, 200, 5
WI = ROWS // NW
BJ = 40
NCH = COLS // BJ


def _sc_lookup(xt_hbm, tbl_hbm, y_hbm, x_v, y_v, tbl_v):
    wid = lax.axis_index("s") * NC + lax.axis_index("c")
    i0 = wid * WI
    pltpu.sync_copy(tbl_hbm, tbl_v)
    t0 = [tbl_v[d] for d in range(D)]
    t1 = [tbl_v[D + d] for d in range(D)]

    def chunk_body(c, carry):
        j0 = c * BJ
        pltpu.sync_copy(xt_hbm.at[pl.ds(j0, BJ), pl.ds(i0, WI)], x_v)

        def vec_body(g, carry2):
            r = g // (WI // L)
            col = (g % (WI // L)) * L
            m = x_v[r, pl.ds(col, L)] == 0
            for d in range(D):
                y_v[d, r, pl.ds(col, L)] = jnp.where(m, t0[d], t1[d])
            return carry2

        lax.fori_loop(0, BJ * (WI // L), vec_body, 0, unroll=4)
        for d in range(D):
            pltpu.sync_copy(y_v.at[d],
                            y_hbm.at[d, pl.ds(j0, BJ), pl.ds(i0, WI)])
        return carry

    lax.fori_loop(0, NCH, chunk_body, 0)


def kernel(x, table):
    xt = x.T
    tbl_splat = jnp.broadcast_to(
        table.reshape(2 * D)[:, None], (2 * D, L)).astype(jnp.float32)
    mesh = plsc.VectorSubcoreMesh(core_axis_name="c", subcore_axis_name="s")
    y = pl.kernel(
        _sc_lookup,
        out_type=jax.ShapeDtypeStruct((D, COLS, ROWS), jnp.float32),
        mesh=mesh,
        compiler_params=pltpu.CompilerParams(needs_layout_passes=False),
        scratch_types=[
            pltpu.VMEM((BJ, WI), jnp.int32),
            pltpu.VMEM((D, BJ, WI), jnp.float32),
            pltpu.VMEM((2 * D, L), jnp.float32),
        ],
    )(xt, tbl_splat)
    return y.transpose(2, 1, 0)


# speedup vs baseline: 1.5045x; 1.2595x over previous
import jax
import jax.numpy as jnp
from jax import lax
from jax.experimental import pallas as pl
from jax.experimental.pallas import tpu as pltpu
from jax.experimental.pallas import tpu_sc as plsc

L = 16
NC, NS = 2, 16
NW = NC * NS
ROWS, COLS, D = 16384, 200, 5
WI = ROWS // NW
BJ = 8
NCH = COLS // BJ
VPC = BJ * (WI // L)


def _sc_lookup(xt_hbm, tbl_hbm, y_hbm, x_v0, x_v1, y_v0, y_v1, tbl_v,
               in_s0, in_s1, out_s0, out_s1):
    wid = lax.axis_index("s") * NC + lax.axis_index("c")
    i0 = wid * WI
    pltpu.sync_copy(tbl_hbm, tbl_v)
    t0 = [tbl_v[d] for d in range(D)]
    t1 = [tbl_v[D + d] for d in range(D)]
    x_bufs, y_bufs = (x_v0, x_v1), (y_v0, y_v1)
    in_sems, out_sems = (in_s0, in_s1), (out_s0, out_s1)

    def x_slice(c):
        return xt_hbm.at[pl.ds(c * BJ, BJ), pl.ds(i0, WI)]

    def y_slice(c):
        return y_hbm.at[:, pl.ds(c * BJ, BJ), pl.ds(i0, WI)]

    def compute(x_v, y_v):
        def vec_body(g, carry):
            r = g // (WI // L)
            col = (g % (WI // L)) * L
            m = x_v[r, pl.ds(col, L)] == 0
            for d in range(D):
                y_v[d, r, pl.ds(col, L)] = jnp.where(m, t0[d], t1[d])
            return carry
        lax.fori_loop(0, VPC, vec_body, 0, unroll=4)

    pltpu.async_copy(x_slice(0), x_v0, in_s0)
    pltpu.async_copy(x_slice(1), x_v1, in_s1)
    for c in range(NCH):
        b = c % 2
        pltpu.make_async_copy(x_slice(c), x_bufs[b], in_sems[b]).wait()
        if c >= 2:
            pltpu.make_async_copy(y_bufs[b], y_slice(c - 2),
                                  out_sems[b]).wait()
        compute(x_bufs[b], y_bufs[b])
        pltpu.async_copy(y_bufs[b], y_slice(c), out_sems[b])
        if c + 2 < NCH:
            pltpu.async_copy(x_slice(c + 2), x_bufs[b], in_sems[b])
    pltpu.make_async_copy(y_bufs[0], y_slice(NCH - 1), out_sems[0]).wait()
    pltpu.make_async_copy(y_bufs[1], y_slice(NCH - 2), out_sems[1]).wait()


def kernel(x, table):
    xt = x.T
    tbl_splat = jnp.broadcast_to(
        table.reshape(2 * D)[:, None], (2 * D, L)).astype(jnp.float32)
    mesh = plsc.VectorSubcoreMesh(core_axis_name="c", subcore_axis_name="s")
    y = pl.kernel(
        _sc_lookup,
        out_type=jax.ShapeDtypeStruct((D, COLS, ROWS), jnp.float32),
        mesh=mesh,
        compiler_params=pltpu.CompilerParams(needs_layout_passes=False),
        scratch_types=[
            pltpu.VMEM((BJ, WI), jnp.int32),
            pltpu.VMEM((BJ, WI), jnp.int32),
            pltpu.VMEM((D, BJ, WI), jnp.float32),
            pltpu.VMEM((D, BJ, WI), jnp.float32),
            pltpu.VMEM((2 * D, L), jnp.float32),
            pltpu.SemaphoreType.DMA,
            pltpu.SemaphoreType.DMA,
            pltpu.SemaphoreType.DMA,
            pltpu.SemaphoreType.DMA,
        ],
    )(xt, tbl_splat)
    return y.transpose(2, 1, 0)


# speedup vs baseline: 2.2388x; 1.4880x over previous
import jax
import jax.numpy as jnp
from jax import lax
from jax.experimental import pallas as pl
from jax.experimental.pallas import tpu as pltpu
from jax.experimental.pallas import tpu_sc as plsc

L = 16
NC, NS = 2, 16
NW = NC * NS
ROWS, COLS, D = 16384, 200, 5
WI = ROWS // NW
BJ = 8
NCH = COLS // BJ
VPC = BJ * (WI // L)


def _sc_lookup(xt_hbm, tbl_hbm, y_hbm, x_v0, x_v1, y_v0, y_v1, tbl_v,
               in_s0, in_s1, out_s0, out_s1):
    wid = lax.axis_index("s") * NC + lax.axis_index("c")
    i0 = wid * WI
    pltpu.sync_copy(tbl_hbm, tbl_v)
    t0 = [tbl_v[d] for d in range(D)]
    t1 = [tbl_v[D + d] for d in range(D)]
    x_bufs, y_bufs = (x_v0, x_v1), (y_v0, y_v1)
    in_sems, out_sems = (in_s0, in_s1), (out_s0, out_s1)

    def x_slice(c):
        return xt_hbm.at[pl.ds(c * BJ, BJ), pl.ds(i0, WI)]

    def y_slice(c):
        return y_hbm.at[:, pl.ds(c * BJ, BJ), pl.ds(i0, WI)]

    def compute(x_v, y_v):
        @plsc.parallel_loop(0, VPC, unroll=4)
        def _vec(g):
            r = g >> 5
            col = (g & 31) * L
            m = x_v[r, pl.ds(col, L)] == 0
            for d in range(D):
                y_v[d, r, pl.ds(col, L)] = jnp.where(m, t0[d], t1[d])

    pltpu.async_copy(x_slice(0), x_v0, in_s0)
    pltpu.async_copy(x_slice(1), x_v1, in_s1)

    def pair(p, carry):
        c0 = 2 * p
        c1 = c0 + 1
        pltpu.make_async_copy(x_slice(c0), x_v0, in_s0).wait()

        @pl.when(p > 0)
        def _():
            pltpu.make_async_copy(y_v0, y_slice(c0 - 2), out_s0).wait()
        compute(x_v0, y_v0)
        pltpu.async_copy(y_v0, y_slice(c0), out_s0)
        pltpu.async_copy(x_slice(c0 + 2), x_v0, in_s0)

        pltpu.make_async_copy(x_slice(c1), x_v1, in_s1).wait()

        @pl.when(p > 0)
        def _():
            pltpu.make_async_copy(y_v1, y_slice(c1 - 2), out_s1).wait()
        compute(x_v1, y_v1)
        pltpu.async_copy(y_v1, y_slice(c1), out_s1)

        @pl.when(p < NCH // 2 - 1)
        def _():
            pltpu.async_copy(x_slice(c1 + 2), x_v1, in_s1)
        return carry

    lax.fori_loop(0, NCH // 2, pair, 0)

    pltpu.make_async_copy(x_slice(NCH - 1), x_v0, in_s0).wait()
    pltpu.make_async_copy(y_v0, y_slice(NCH - 3), out_s0).wait()
    compute(x_v0, y_v0)
    pltpu.async_copy(y_v0, y_slice(NCH - 1), out_s0)

    pltpu.make_async_copy(y_v1, y_slice(NCH - 2), out_s1).wait()
    pltpu.make_async_copy(y_v0, y_slice(NCH - 1), out_s0).wait()


def kernel(x, table):
    xt = x.T
    tbl_splat = jnp.broadcast_to(
        table.reshape(2 * D)[:, None], (2 * D, L)).astype(jnp.float32)
    mesh = plsc.VectorSubcoreMesh(core_axis_name="c", subcore_axis_name="s")
    y = pl.kernel(
        _sc_lookup,
        out_type=jax.ShapeDtypeStruct((D, COLS, ROWS), jnp.float32),
        mesh=mesh,
        compiler_params=pltpu.CompilerParams(needs_layout_passes=False),
        scratch_types=[
            pltpu.VMEM((BJ, WI), jnp.int32),
            pltpu.VMEM((BJ, WI), jnp.int32),
            pltpu.VMEM((D, BJ, WI), jnp.float32),
            pltpu.VMEM((D, BJ, WI), jnp.float32),
            pltpu.VMEM((2 * D, L), jnp.float32),
            pltpu.SemaphoreType.DMA,
            pltpu.SemaphoreType.DMA,
            pltpu.SemaphoreType.DMA,
            pltpu.SemaphoreType.DMA,
        ],
    )(xt, tbl_splat)
    return y.transpose(2, 1, 0)
